# baseline (speedup 1.0000x reference)
import functools

import jax
import jax.numpy as jnp
from jax import lax
from jax.experimental import pallas as pl
from jax.experimental.pallas import tpu as pltpu
from jax.experimental.pallas import tpu_sc as plsc

_NC = 2
_NS = 16
_NW = _NC * _NS
_EB = 125
_NUM_GRAPHS = 128


def _make_agg(n, d, e):
  nb = e // _EB
  npw = nb // _NW
  rps = (n // _NS) // 8 * 8
  tail = n - _NS * rps

  mesh = plsc.VectorSubcoreMesh(core_axis_name="c", subcore_axis_name="s")

  ch = npw // 2

  @functools.partial(
      pl.kernel,
      mesh=mesh,
      out_type=jax.ShapeDtypeStruct((_NC * n, d), jnp.float32),
      scratch_types=[
          pltpu.VMEM((ch, _EB), jnp.int32),
          pltpu.VMEM((ch, _EB), jnp.int32),
          pltpu.VMEM((_EB, d), jnp.float32),
          pltpu.VMEM((_EB, d), jnp.float32),
          pltpu.VMEM_SHARED((n, d), jnp.float32),
          pltpu.SemaphoreType.DMA,
          pltpu.SemaphoreType.DMA,
          pltpu.SemaphoreType.DMA,
          pltpu.SemaphoreType.DMA,
      ],
  )
  def agg(h_hbm, src_hbm, dst_hbm, out_hbm, src_v, dst_v, rows0, rows1,
          acc_sh, gsem0, gsem1, ssem0, ssem1):
    c = lax.axis_index("c")
    s = lax.axis_index("s")
    wid = s * _NC + c
    r0 = s * rps
    pltpu.sync_copy(h_hbm.at[pl.ds(r0, rps)], acc_sh.at[pl.ds(r0, rps)])

    @pl.when(s == 0)
    def _():
      pltpu.sync_copy(h_hbm.at[pl.ds(_NS * rps, tail)],
                      acc_sh.at[pl.ds(_NS * rps, tail)])

    plsc.subcore_barrier()

    for ph in range(npw // ch):
      pltpu.sync_copy(src_hbm.at[pl.ds(wid * npw + ph * ch, ch)], src_v)
      pltpu.sync_copy(dst_hbm.at[pl.ds(wid * npw + ph * ch, ch)], dst_v)
      pltpu.async_copy(h_hbm.at[src_v.at[0]], rows0, gsem0)
      pltpu.async_copy(h_hbm.at[src_v.at[1]], rows1, gsem1)

      def body(i, carry):
        b0 = 2 * i
        b1 = b0 + 1
        pltpu.make_async_copy(h_hbm.at[src_v.at[b0]], rows0, gsem0).wait()
        pltpu.async_copy(rows0, acc_sh.at[dst_v.at[b0]], ssem0, add=True)
        pltpu.make_async_copy(h_hbm.at[src_v.at[b1]], rows1, gsem1).wait()
        pltpu.async_copy(rows1, acc_sh.at[dst_v.at[b1]], ssem1, add=True)
        pltpu.make_async_copy(rows0, acc_sh.at[dst_v.at[b0]], ssem0).wait()

        @pl.when(b0 + 2 < ch)
        def _():
          pltpu.async_copy(h_hbm.at[src_v.at[b0 + 2]], rows0, gsem0)

        pltpu.make_async_copy(rows1, acc_sh.at[dst_v.at[b1]], ssem1).wait()

        @pl.when(b1 + 2 < ch)
        def _():
          pltpu.async_copy(h_hbm.at[src_v.at[b1 + 2]], rows1, gsem1)

        return carry

      lax.fori_loop(0, ch // 2, body, 0)
    plsc.subcore_barrier()
    pltpu.sync_copy(acc_sh.at[pl.ds(r0, rps)],
                    out_hbm.at[pl.ds(c * n + r0, rps)])

    @pl.when(s == 0)
    def _():
      pltpu.sync_copy(acc_sh.at[pl.ds(_NS * rps, tail)],
                      out_hbm.at[pl.ds(c * n + _NS * rps, tail)])

  return agg


def _bn_in(y, g, b):
  mu = jnp.mean(y, axis=0, keepdims=True)
  var = jnp.mean((y - mu) * (y - mu), axis=0, keepdims=True)
  return (y - mu) / jnp.sqrt(var + 1e-5) * g + b


def _mlp_core(n, h_ref, parts_ref, w1_ref, b1_ref, g1_ref, be1_ref,
              w2_ref, b2_ref, g2_ref, be2_ref):
  z = parts_ref[0:n, :] + parts_ref[n:2 * n, :] - h_ref[...]
  y = jnp.maximum(jnp.dot(z, w1_ref[...], preferred_element_type=jnp.float32)
                  + b1_ref[...], 0.0)
  y = _bn_in(y, g1_ref[...], be1_ref[...])
  y = jnp.maximum(jnp.dot(y, w2_ref[...], preferred_element_type=jnp.float32)
                  + b2_ref[...], 0.0)
  return _bn_in(y, g2_ref[...], be2_ref[...])


def _make_mlp_mid(n, d):
  def body(h_ref, parts_ref, w1_ref, b1_ref, g1_ref, be1_ref,
           w2_ref, b2_ref, g2_ref, be2_ref, bng_ref, bnb_ref, out_ref):
    y = _mlp_core(n, h_ref, parts_ref, w1_ref, b1_ref, g1_ref, be1_ref,
                  w2_ref, b2_ref, g2_ref, be2_ref)
    out_ref[...] = _bn_in(y, bng_ref[...], bnb_ref[...])

  return pl.pallas_call(
      body, out_shape=jax.ShapeDtypeStruct((n, d), jnp.float32))


def _make_mlp_final(n, d, g, t):
  def body(h_ref, parts_ref, w1_ref, b1_ref, g1_ref, be1_ref,
           w2_ref, b2_ref, g2_ref, be2_ref, batch_ref, wl_ref, bl_ref,
           out_ref):
    y = _mlp_core(n, h_ref, parts_ref, w1_ref, b1_ref, g1_ref, be1_ref,
                  w2_ref, b2_ref, g2_ref, be2_ref)
    seg = lax.broadcasted_iota(jnp.int32, (g, n), 0)
    oh = (seg == batch_ref[...]).astype(jnp.float32)
    pooled = jnp.dot(oh, y, preferred_element_type=jnp.float32)
    out_ref[...] = (jnp.dot(pooled, wl_ref[...],
                            preferred_element_type=jnp.float32) + bl_ref[...])

  return pl.pallas_call(
      body, out_shape=jax.ShapeDtypeStruct((g, t), jnp.float32))


def kernel(x, edge_index, batch, W1s, b1s, g1s, be1s, W2s, b2s, g2s, be2s,
           bn_g, bn_b, Wl, bl):
  n, d = x.shape
  e = edge_index.shape[1]
  num_layers = W1s.shape[0]
  h_dim = W1s.shape[2]
  t = Wl.shape[1]

  src2d = edge_index[0].reshape(e // _EB, _EB)
  dst2d = edge_index[1].reshape(e // _EB, _EB)
  batch2d = batch.reshape(1, n)

  agg = _make_agg(n, d, e)
  mlp_mid = _make_mlp_mid(n, h_dim)
  mlp_final = _make_mlp_final(n, h_dim, _NUM_GRAPHS, t)

  r = lambda v: v.reshape(1, -1)
  h = x
  for i in range(num_layers):
    parts = agg(h, src2d, dst2d)
    if i < num_layers - 1:
      h = mlp_mid(h, parts, W1s[i], r(b1s[i]), r(g1s[i]), r(be1s[i]),
                  W2s[i], r(b2s[i]), r(g2s[i]), r(be2s[i]),
                  r(bn_g[i]), r(bn_b[i]))
    else:
      out = mlp_final(h, parts, W1s[i], r(b1s[i]), r(g1s[i]), r(be1s[i]),
                      W2s[i], r(b2s[i]), r(g2s[i]), r(be2s[i]),
                      batch2d, Wl, r(bl))
  return out


# speedup vs baseline: 1.2543x; 1.2543x over previous
import functools

import jax
import jax.numpy as jnp
from jax import lax
from jax.experimental import pallas as pl
from jax.experimental.pallas import tpu as pltpu
from jax.experimental.pallas import tpu_sc as plsc

_NC = 2
_NS = 16
_NW = _NC * _NS
_EB = 125
_NUM_GRAPHS = 128


def _make_agg(n, d, e):
  nb = e // _EB
  npw = nb // _NW
  rps = (n // _NS) // 8 * 8
  tail = n - _NS * rps

  mesh = plsc.VectorSubcoreMesh(core_axis_name="c", subcore_axis_name="s")

  ch = npw // 2

  @functools.partial(
      pl.kernel,
      mesh=mesh,
      out_type=jax.ShapeDtypeStruct((_NC * n, d), jnp.float32),
      scratch_types=[
          pltpu.VMEM((ch, _EB), jnp.int32),
          pltpu.VMEM((ch, _EB), jnp.int32),
          pltpu.VMEM((_EB, d), jnp.float32),
          pltpu.VMEM((_EB, d), jnp.float32),
          pltpu.VMEM_SHARED((n, d), jnp.float32),
          pltpu.SemaphoreType.DMA,
          pltpu.SemaphoreType.DMA,
          pltpu.SemaphoreType.DMA,
          pltpu.SemaphoreType.DMA,
      ],
  )
  def agg(h_hbm, src_hbm, dst_hbm, out_hbm, src_v, dst_v, rows0, rows1,
          acc_sh, gsem0, gsem1, ssem0, ssem1):
    c = lax.axis_index("c")
    s = lax.axis_index("s")
    wid = s * _NC + c
    r0 = s * rps
    pltpu.sync_copy(h_hbm.at[pl.ds(r0, rps)], acc_sh.at[pl.ds(r0, rps)])

    @pl.when(s == 0)
    def _():
      pltpu.sync_copy(h_hbm.at[pl.ds(_NS * rps, tail)],
                      acc_sh.at[pl.ds(_NS * rps, tail)])

    plsc.subcore_barrier()

    for ph in range(npw // ch):
      pltpu.sync_copy(src_hbm.at[pl.ds(wid * npw + ph * ch, ch)], src_v)
      pltpu.sync_copy(dst_hbm.at[pl.ds(wid * npw + ph * ch, ch)], dst_v)
      pltpu.async_copy(h_hbm.at[src_v.at[0]], rows0, gsem0)
      pltpu.async_copy(h_hbm.at[src_v.at[1]], rows1, gsem1)

      def body(i, carry):
        b0 = 2 * i
        b1 = b0 + 1
        pltpu.make_async_copy(h_hbm.at[src_v.at[b0]], rows0, gsem0).wait()
        pltpu.sync_copy(rows0, acc_sh.at[dst_v.at[b0]], add=True)

        @pl.when(b0 + 2 < ch)
        def _():
          pltpu.async_copy(h_hbm.at[src_v.at[b0 + 2]], rows0, gsem0)

        pltpu.make_async_copy(h_hbm.at[src_v.at[b1]], rows1, gsem1).wait()
        pltpu.sync_copy(rows1, acc_sh.at[dst_v.at[b1]], add=True)

        @pl.when(b1 + 2 < ch)
        def _():
          pltpu.async_copy(h_hbm.at[src_v.at[b1 + 2]], rows1, gsem1)

        return carry

      lax.fori_loop(0, ch // 2, body, 0)
    plsc.subcore_barrier()
    pltpu.sync_copy(acc_sh.at[pl.ds(r0, rps)],
                    out_hbm.at[pl.ds(c * n + r0, rps)])

    @pl.when(s == 0)
    def _():
      pltpu.sync_copy(acc_sh.at[pl.ds(_NS * rps, tail)],
                      out_hbm.at[pl.ds(c * n + _NS * rps, tail)])

  return agg


def _bn_in(y, g, b):
  mu = jnp.mean(y, axis=0, keepdims=True)
  var = jnp.mean((y - mu) * (y - mu), axis=0, keepdims=True)
  return (y - mu) / jnp.sqrt(var + 1e-5) * g + b


def _mlp_core(n, h_ref, parts_ref, w1_ref, b1_ref, g1_ref, be1_ref,
              w2_ref, b2_ref, g2_ref, be2_ref):
  z = parts_ref[0:n, :] + parts_ref[n:2 * n, :] - h_ref[...]
  y = jnp.maximum(jnp.dot(z, w1_ref[...], preferred_element_type=jnp.float32)
                  + b1_ref[...], 0.0)
  y = _bn_in(y, g1_ref[...], be1_ref[...])
  y = jnp.maximum(jnp.dot(y, w2_ref[...], preferred_element_type=jnp.float32)
                  + b2_ref[...], 0.0)
  return _bn_in(y, g2_ref[...], be2_ref[...])


def _make_mlp_mid(n, d):
  def body(h_ref, parts_ref, w1_ref, b1_ref, g1_ref, be1_ref,
           w2_ref, b2_ref, g2_ref, be2_ref, bng_ref, bnb_ref, out_ref):
    y = _mlp_core(n, h_ref, parts_ref, w1_ref, b1_ref, g1_ref, be1_ref,
                  w2_ref, b2_ref, g2_ref, be2_ref)
    out_ref[...] = _bn_in(y, bng_ref[...], bnb_ref[...])

  return pl.pallas_call(
      body, out_shape=jax.ShapeDtypeStruct((n, d), jnp.float32))


def _make_mlp_final(n, d, g, t):
  def body(h_ref, parts_ref, w1_ref, b1_ref, g1_ref, be1_ref,
           w2_ref, b2_ref, g2_ref, be2_ref, batch_ref, wl_ref, bl_ref,
           out_ref):
    y = _mlp_core(n, h_ref, parts_ref, w1_ref, b1_ref, g1_ref, be1_ref,
                  w2_ref, b2_ref, g2_ref, be2_ref)
    seg = lax.broadcasted_iota(jnp.int32, (g, n), 0)
    oh = (seg == batch_ref[...]).astype(jnp.float32)
    pooled = jnp.dot(oh, y, preferred_element_type=jnp.float32)
    out_ref[...] = (jnp.dot(pooled, wl_ref[...],
                            preferred_element_type=jnp.float32) + bl_ref[...])

  return pl.pallas_call(
      body, out_shape=jax.ShapeDtypeStruct((g, t), jnp.float32))


def kernel(x, edge_index, batch, W1s, b1s, g1s, be1s, W2s, b2s, g2s, be2s,
           bn_g, bn_b, Wl, bl):
  n, d = x.shape
  e = edge_index.shape[1]
  num_layers = W1s.shape[0]
  h_dim = W1s.shape[2]
  t = Wl.shape[1]

  src2d = edge_index[0].reshape(e // _EB, _EB)
  dst2d = edge_index[1].reshape(e // _EB, _EB)
  batch2d = batch.reshape(1, n)

  agg = _make_agg(n, d, e)
  mlp_mid = _make_mlp_mid(n, h_dim)
  mlp_final = _make_mlp_final(n, h_dim, _NUM_GRAPHS, t)

  r = lambda v: v.reshape(1, -1)
  h = x
  for i in range(num_layers):
    parts = agg(h, src2d, dst2d)
    if i < num_layers - 1:
      h = mlp_mid(h, parts, W1s[i], r(b1s[i]), r(g1s[i]), r(be1s[i]),
                  W2s[i], r(b2s[i]), r(g2s[i]), r(be2s[i]),
                  r(bn_g[i]), r(bn_b[i]))
    else:
      out = mlp_final(h, parts, W1s[i], r(b1s[i]), r(g1s[i]), r(be1s[i]),
                      W2s[i], r(b2s[i]), r(g2s[i]), r(be2s[i]),
                      batch2d, Wl, r(bl))
  return out
